# Initial kernel scaffold; baseline (speedup 1.0000x reference)
import jax
import jax.numpy as jnp
from jax.experimental import pallas as pl


def kernel(z, dictionary):
    raise NotImplementedError("write your pallas kernel here")


# speedup vs baseline: 3.6528x; 3.6528x over previous
import jax
import jax.numpy as jnp
from jax.experimental import pallas as pl

_NUM_EMBEDDINGS = 8192
_EMBED_DIM = 32
_SPARSITY = 5
_COMMIT = 0.25
_EPS = 1e-10
_TB = 256


def _body(x_ref, d_ref, q_ref, loss_ref):
    D = d_ref[...]
    norm = jnp.sqrt(jnp.sum(D * D, axis=0, keepdims=True))
    Dn = D / (norm + _EPS)
    x = x_ref[...]
    corr = jnp.dot(x, Dn)
    a = jnp.abs(corr)
    iota = jax.lax.broadcasted_iota(jnp.int32, a.shape, 1)
    coef = jnp.zeros_like(corr)
    for _ in range(_SPARSITY):
        m = jnp.max(a, axis=1, keepdims=True)
        idx = jnp.min(jnp.where(a == m, iota, _NUM_EMBEDDINGS),
                      axis=1, keepdims=True)
        sel = iota == idx
        coef = jnp.where(sel, corr, coef)
        a = jnp.where(sel, -1.0, a)
    recon = jnp.dot(coef, Dn.T)
    diff = recon - x
    q_ref[...] = x + diff

    @pl.when(pl.program_id(0) == 0)
    def _init():
        loss_ref[...] = jnp.zeros_like(loss_ref)

    loss_ref[...] += jnp.sum(diff * diff).reshape(1, 1)


def kernel(z, dictionary):
    B, T, C = z.shape
    X = z.reshape(-1, C)
    Bt = X.shape[0]
    num_blocks = Bt // _TB
    q, losssum = pl.pallas_call(
        _body,
        grid=(num_blocks,),
        in_specs=[
            pl.BlockSpec((_TB, C), lambda i: (i, 0)),
            pl.BlockSpec((C, _NUM_EMBEDDINGS), lambda i: (0, 0)),
        ],
        out_specs=[
            pl.BlockSpec((_TB, C), lambda i: (i, 0)),
            pl.BlockSpec((1, 1), lambda i: (0, 0)),
        ],
        out_shape=[
            jax.ShapeDtypeStruct((Bt, C), jnp.float32),
            jax.ShapeDtypeStruct((1, 1), jnp.float32),
        ],
    )(X, dictionary)
    loss = (1.0 + _COMMIT) * losssum[0, 0] / (Bt * C)
    return q.reshape(B, T, C), loss


# speedup vs baseline: 4.4019x; 1.2051x over previous
import jax
import jax.numpy as jnp
from jax.experimental import pallas as pl

_NUM_EMBEDDINGS = 8192
_EMBED_DIM = 32
_SPARSITY = 5
_COMMIT = 0.25
_EPS = 1e-10
_TB = 512


def _body(x_ref, d_ref, q_ref, loss_ref):
    D = d_ref[...]
    norm = jnp.sqrt(jnp.sum(D * D, axis=0, keepdims=True))
    Dn = D / (norm + _EPS)
    x = x_ref[...]
    corr = jnp.dot(x, Dn)
    a = jnp.abs(corr)
    iota = jax.lax.broadcasted_iota(jnp.int32, a.shape, 1).astype(jnp.float32)
    for _ in range(_SPARSITY):
        m = jnp.max(a, axis=1, keepdims=True)
        idx = jnp.min(jnp.where(a == m, iota, float(_NUM_EMBEDDINGS)),
                      axis=1, keepdims=True)
        a = jnp.where(iota == idx, -1.0, a)
    coef = jnp.where(a < 0, corr, 0.0)
    recon = jnp.dot(coef, Dn.T)
    diff = recon - x
    q_ref[...] = x + diff

    @pl.when(pl.program_id(0) == 0)
    def _init():
        loss_ref[...] = jnp.zeros_like(loss_ref)

    loss_ref[...] += jnp.sum(diff * diff).reshape(1, 1)


def kernel(z, dictionary):
    B, T, C = z.shape
    X = z.reshape(-1, C)
    Bt = X.shape[0]
    num_blocks = Bt // _TB
    q, losssum = pl.pallas_call(
        _body,
        grid=(num_blocks,),
        in_specs=[
            pl.BlockSpec((_TB, C), lambda i: (i, 0)),
            pl.BlockSpec((C, _NUM_EMBEDDINGS), lambda i: (0, 0)),
        ],
        out_specs=[
            pl.BlockSpec((_TB, C), lambda i: (i, 0)),
            pl.BlockSpec((1, 1), lambda i: (0, 0)),
        ],
        out_shape=[
            jax.ShapeDtypeStruct((Bt, C), jnp.float32),
            jax.ShapeDtypeStruct((1, 1), jnp.float32),
        ],
    )(X, dictionary)
    loss = (1.0 + _COMMIT) * losssum[0, 0] / (Bt * C)
    return q.reshape(B, T, C), loss


# speedup vs baseline: 4.8812x; 1.1089x over previous
import jax
import jax.numpy as jnp
from jax.experimental import pallas as pl
from jax.experimental.pallas import tpu as pltpu

_NUM_EMBEDDINGS = 8192
_EMBED_DIM = 32
_SPARSITY = 5
_COMMIT = 0.25
_EPS = 1e-10
_TB = 256


def _body(xp_ref, x_ref, d_ref, q_ref, loss_ref, corr_ref, dn_ref):
    i = pl.program_id(0)
    nb = pl.num_programs(0)

    @pl.when(i == 0)
    def _prologue():
        D = d_ref[...]
        norm = jnp.sqrt(jnp.sum(D * D, axis=0, keepdims=True))
        dn_ref[...] = D / (norm + _EPS)
        loss_ref[...] = jnp.zeros_like(loss_ref)
        corr_ref[0] = jnp.dot(x_ref[...], dn_ref[...])

    Dn = dn_ref[...]

    @pl.when(i + 1 < nb)
    def _prefetch():
        corr_ref[(i + 1) % 2] = jnp.dot(xp_ref[...], Dn)

    corr = corr_ref[i % 2]
    a = jnp.abs(corr)
    iota = jax.lax.broadcasted_iota(jnp.int32, a.shape, 1).astype(jnp.float32)
    for _ in range(_SPARSITY):
        m = jnp.max(a, axis=1, keepdims=True)
        idx = jnp.min(jnp.where(a == m, iota, float(_NUM_EMBEDDINGS)),
                      axis=1, keepdims=True)
        a = jnp.where(iota == idx, -1.0, a)
    coef = jnp.where(a < 0, corr, 0.0)
    recon = jnp.dot(coef, Dn.T)
    x = x_ref[...]
    diff = recon - x
    q_ref[...] = x + diff
    loss_ref[...] += jnp.sum(diff * diff).reshape(1, 1)


def kernel(z, dictionary):
    B, T, C = z.shape
    X = z.reshape(-1, C)
    Bt = X.shape[0]
    nb = Bt // _TB
    q, losssum = pl.pallas_call(
        _body,
        grid=(nb,),
        in_specs=[
            pl.BlockSpec((_TB, C), lambda i: ((i + 1) % nb, 0)),
            pl.BlockSpec((_TB, C), lambda i: (i, 0)),
            pl.BlockSpec((C, _NUM_EMBEDDINGS), lambda i: (0, 0)),
        ],
        out_specs=[
            pl.BlockSpec((_TB, C), lambda i: (i, 0)),
            pl.BlockSpec((1, 1), lambda i: (0, 0)),
        ],
        out_shape=[
            jax.ShapeDtypeStruct((Bt, C), jnp.float32),
            jax.ShapeDtypeStruct((1, 1), jnp.float32),
        ],
        scratch_shapes=[
            pltpu.VMEM((2, _TB, _NUM_EMBEDDINGS), jnp.float32),
            pltpu.VMEM((_EMBED_DIM, _NUM_EMBEDDINGS), jnp.float32),
        ],
    )(X, X, dictionary)
    loss = (1.0 + _COMMIT) * losssum[0, 0] / (Bt * C)
    return q.reshape(B, T, C), loss
